# baseline (speedup 1.0000x reference)
import functools

import jax
import jax.numpy as jnp
from jax import lax
from jax.experimental import pallas as pl
from jax.experimental.pallas import tpu as pltpu
from jax.experimental.pallas import tpu_sc as plsc

LANES = 16
CHUNK = 128
UNROLL = 8
EPS = 1e-5


def _rsqrt_vec(x):
    i = plsc.bitcast(x, jnp.int32)
    i = jnp.int32(0x5F3759DF) - lax.shift_right_logical(i, 1)
    y = plsc.bitcast(i, jnp.float32)
    hx = 0.5 * x
    for _ in range(3):
        y = y * (1.5 - hx * y * y)
    return y


def _make_sc_kernel(rows, hidden, max_pos, seq):
    n_workers = 32
    rpw = rows // n_workers
    n_chunks = rpw // CHUNK
    nk = hidden // LANES
    inv_h = 1.0 / hidden

    mesh = plsc.VectorSubcoreMesh(core_axis_name="c", subcore_axis_name="s")

    @functools.partial(
        pl.kernel,
        mesh=mesh,
        out_type=jax.ShapeDtypeStruct((rows, hidden), jnp.float32),
        compiler_params=pltpu.CompilerParams(
            needs_layout_passes=False, use_tc_tiling_on_sc=False),
        scratch_types=[
            pltpu.VMEM((CHUNK,), jnp.int32),
            pltpu.VMEM((CHUNK,), jnp.int32),
            pltpu.VMEM((CHUNK, hidden), jnp.float32),
            pltpu.VMEM((CHUNK, hidden), jnp.float32),
            pltpu.VMEM((max_pos, hidden), jnp.float32),
            pltpu.VMEM((hidden,), jnp.float32),
            pltpu.VMEM((hidden,), jnp.float32),
            pltpu.SemaphoreType.DMA,
            pltpu.SemaphoreType.DMA,
        ],
    )
    def sc_kernel(ids_hbm, table_hbm, pos_hbm, gam_hbm, bet_hbm, out_hbm,
                  idx_v0, idx_v1, rows_v0, rows_v1, pos_v, gam_v, bet_v,
                  sem0, sem1):
        wid = lax.axis_index("s") * 2 + lax.axis_index("c")
        base = wid * rpw
        pltpu.sync_copy(pos_hbm, pos_v)
        pltpu.sync_copy(gam_hbm, gam_v)
        pltpu.sync_copy(bet_hbm, bet_v)
        idx_bufs = (idx_v0, idx_v1)
        row_bufs = (rows_v0, rows_v1)
        sems = (sem0, sem1)
        gs = [gam_v[pl.ds(k * LANES, LANES)] for k in range(nk)]
        bs = [bet_v[pl.ds(k * LANES, LANES)] for k in range(nk)]
        iota16 = lax.iota(jnp.int32, LANES)
        perms = [iota16 ^ d for d in (1, 2, 4, 8)]

        def _lane_sum(v):
            for pm in perms:
                v = v + v.at[pm].get(mode="promise_in_bounds")
            return v

        def start_gather(c, slot):
            r0 = base + c * CHUNK
            pltpu.sync_copy(ids_hbm.at[pl.ds(r0, CHUNK)], idx_bufs[slot])
            pltpu.async_copy(table_hbm.at[idx_bufs[slot]], row_bufs[slot],
                             sems[slot])

        def process(c, slot):
            r0 = base + c * CHUNK
            rows_v = row_bufs[slot]
            pltpu.make_async_copy(table_hbm.at[idx_bufs[slot]], rows_v,
                                  sems[slot]).wait()

            @plsc.parallel_loop(0, CHUNK, unroll=UNROLL)
            def row_body(r):
                p = lax.rem(r0 + r, seq)
                xs = []
                for k in range(nk):
                    t = rows_v[r, pl.ds(k * LANES, LANES)]
                    pe = pos_v[p, pl.ds(k * LANES, LANES)]
                    xs.append(t + pe)
                s = (xs[0] + xs[1]) + (xs[2] + xs[3])
                sq = (xs[0] * xs[0] + xs[1] * xs[1]) + \
                     (xs[2] * xs[2] + xs[3] * xs[3])
                mean = _lane_sum(s) * inv_h
                var = _lane_sum(sq) * inv_h - mean * mean
                rstd = _rsqrt_vec(var + EPS)
                nm = mean * rstd
                for k in range(nk):
                    y = (xs[k] * rstd - nm) * gs[k] + bs[k]
                    rows_v[r, pl.ds(k * LANES, LANES)] = y

            pltpu.sync_copy(rows_v, out_hbm.at[pl.ds(r0, CHUNK)])

        start_gather(jnp.int32(0), 0)

        def pair_body(i, carry):
            c0 = 2 * i
            start_gather(c0 + 1, 1)
            process(c0, 0)

            @pl.when(c0 + 2 < n_chunks)
            def _():
                start_gather(c0 + 2, 0)

            process(c0 + 1, 1)
            return carry

        lax.fori_loop(0, n_chunks // 2, pair_body, 0)

    return sc_kernel


def kernel(input_ids, token_table, pos_table, ln_gamma, ln_beta):
    b, s = input_ids.shape
    vocab, hidden = token_table.shape
    max_pos = pos_table.shape[0]
    rows = b * s
    ids = input_ids.reshape(rows).astype(jnp.int32)
    sc = _make_sc_kernel(rows, hidden, max_pos, s)
    out = sc(ids, token_table, pos_table, ln_gamma, ln_beta)
    return out.reshape(b, s, hidden)


# speedup vs baseline: 1.1116x; 1.1116x over previous
import functools

import jax
import jax.numpy as jnp
from jax import lax
from jax.experimental import pallas as pl
from jax.experimental.pallas import tpu as pltpu
from jax.experimental.pallas import tpu_sc as plsc

LANES = 16
CHUNK = 128
UNROLL = 4
EPS = 1e-5


def _rsqrt_vec(x):
    i = plsc.bitcast(x, jnp.int32)
    i = jnp.int32(0x5F3759DF) - lax.shift_right_logical(i, 1)
    y = plsc.bitcast(i, jnp.float32)
    hx = 0.5 * x
    for _ in range(3):
        y = y * (1.5 - hx * y * y)
    return y


def _rsqrt_vec2(x):
    i = plsc.bitcast(x, jnp.int32)
    i = jnp.int32(0x5F3759DF) - lax.shift_right_logical(i, 1)
    y = plsc.bitcast(i, jnp.float32)
    hx = 0.5 * x
    for _ in range(2):
        y = y * (1.5 - hx * y * y)
    return y


def _make_sc_kernel(rows, hidden, max_pos, seq):
    n_workers = 32
    rpw = rows // n_workers
    n_chunks = rpw // CHUNK
    nk = hidden // LANES
    inv_h = 1.0 / hidden

    mesh = plsc.VectorSubcoreMesh(core_axis_name="c", subcore_axis_name="s")

    @functools.partial(
        pl.kernel,
        mesh=mesh,
        out_type=jax.ShapeDtypeStruct((rows, hidden), jnp.float32),
        compiler_params=pltpu.CompilerParams(
            needs_layout_passes=False, use_tc_tiling_on_sc=False),
        scratch_types=[
            pltpu.VMEM((CHUNK,), jnp.int32),
            pltpu.VMEM((CHUNK,), jnp.int32),
            pltpu.VMEM((CHUNK, hidden), jnp.float32),
            pltpu.VMEM((CHUNK, hidden), jnp.float32),
            pltpu.VMEM((max_pos, hidden), jnp.float32),
            pltpu.VMEM((hidden,), jnp.float32),
            pltpu.VMEM((hidden,), jnp.float32),
            pltpu.SemaphoreType.DMA,
            pltpu.SemaphoreType.DMA,
        ],
    )
    def sc_kernel(ids_hbm, table_hbm, pos_hbm, gam_hbm, bet_hbm, out_hbm,
                  idx_v0, idx_v1, rows_v0, rows_v1, pos_v, gam_v, bet_v,
                  sem0, sem1):
        wid = lax.axis_index("s") * 2 + lax.axis_index("c")
        base = wid * rpw
        pltpu.sync_copy(pos_hbm, pos_v)
        pltpu.sync_copy(gam_hbm, gam_v)
        pltpu.sync_copy(bet_hbm, bet_v)
        idx_bufs = (idx_v0, idx_v1)
        row_bufs = (rows_v0, rows_v1)
        sems = (sem0, sem1)
        gs = [gam_v[pl.ds(k * LANES, LANES)] for k in range(nk)]
        bs = [bet_v[pl.ds(k * LANES, LANES)] for k in range(nk)]
        iota16 = lax.iota(jnp.int32, LANES)
        perms = [iota16 ^ d for d in (1, 2, 4, 8)]

        def _lane_sum(v):
            for pm in perms:
                v = v + v.at[pm].get(mode="promise_in_bounds")
            return v

        def start_gather(c, slot):
            r0 = base + c * CHUNK
            pltpu.sync_copy(ids_hbm.at[pl.ds(r0, CHUNK)], idx_bufs[slot])
            pltpu.async_copy(table_hbm.at[idx_bufs[slot]], row_bufs[slot],
                             sems[slot])

        def process(c, slot):
            r0 = base + c * CHUNK
            rows_v = row_bufs[slot]
            pltpu.make_async_copy(table_hbm.at[idx_bufs[slot]], rows_v,
                                  sems[slot]).wait()

            @plsc.parallel_loop(0, CHUNK, unroll=UNROLL)
            def row_body(r):
                p = lax.rem(r0 + r, seq)
                xs = []
                for k in range(nk):
                    t = rows_v[r, pl.ds(k * LANES, LANES)]
                    pe = pos_v[p, pl.ds(k * LANES, LANES)]
                    xs.append(t + pe)
                s = (xs[0] + xs[1]) + (xs[2] + xs[3])
                sq = (xs[0] * xs[0] + xs[1] * xs[1]) + \
                     (xs[2] * xs[2] + xs[3] * xs[3])
                mean = _lane_sum(s) * inv_h
                var = _lane_sum(sq) * inv_h - mean * mean
                rstd = _rsqrt_vec2(var + EPS)
                nm = mean * rstd
                for k in range(nk):
                    y = (xs[k] * rstd - nm) * gs[k] + bs[k]
                    rows_v[r, pl.ds(k * LANES, LANES)] = y

            pltpu.sync_copy(rows_v, out_hbm.at[pl.ds(r0, CHUNK)])

        start_gather(jnp.int32(0), 0)

        def pair_body(i, carry):
            c0 = 2 * i
            start_gather(c0 + 1, 1)
            process(c0, 0)

            @pl.when(c0 + 2 < n_chunks)
            def _():
                start_gather(c0 + 2, 0)

            process(c0 + 1, 1)
            return carry

        lax.fori_loop(0, n_chunks // 2, pair_body, 0)

    return sc_kernel


def kernel(input_ids, token_table, pos_table, ln_gamma, ln_beta):
    b, s = input_ids.shape
    vocab, hidden = token_table.shape
    max_pos = pos_table.shape[0]
    rows = b * s
    ids = input_ids.reshape(rows).astype(jnp.int32)
    sc = _make_sc_kernel(rows, hidden, max_pos, s)
    out = sc(ids, token_table, pos_table, ln_gamma, ln_beta)
    return out.reshape(b, s, hidden)


# speedup vs baseline: 1.1610x; 1.0445x over previous
import functools

import jax
import jax.numpy as jnp
from jax import lax
from jax.experimental import pallas as pl
from jax.experimental.pallas import tpu as pltpu
from jax.experimental.pallas import tpu_sc as plsc

LANES = 16
CHUNK = 128
UNROLL = 4
EPS = 1e-5


def _rsqrt_vec(x):
    i = plsc.bitcast(x, jnp.int32)
    i = jnp.int32(0x5F3759DF) - lax.shift_right_logical(i, 1)
    y = plsc.bitcast(i, jnp.float32)
    hx = 0.5 * x
    for _ in range(3):
        y = y * (1.5 - hx * y * y)
    return y


def _rsqrt_vec2(x):
    i = plsc.bitcast(x, jnp.int32)
    i = jnp.int32(0x5F3759DF) - lax.shift_right_logical(i, 1)
    y = plsc.bitcast(i, jnp.float32)
    hx = 0.5 * x
    for _ in range(2):
        y = y * (1.5 - hx * y * y)
    return y


def _make_sc_kernel(rows, hidden, max_pos, seq):
    n_workers = 32
    rpw = rows // n_workers
    n_chunks = rpw // CHUNK
    nk = hidden // LANES
    inv_h = 1.0 / hidden

    mesh = plsc.VectorSubcoreMesh(core_axis_name="c", subcore_axis_name="s")

    @functools.partial(
        pl.kernel,
        mesh=mesh,
        out_type=jax.ShapeDtypeStruct((rows, hidden), jnp.float32),
        compiler_params=pltpu.CompilerParams(
            needs_layout_passes=False, use_tc_tiling_on_sc=False),
        scratch_types=[
            pltpu.VMEM((CHUNK,), jnp.int32),
            pltpu.VMEM((CHUNK,), jnp.int32),
            pltpu.VMEM((CHUNK,), jnp.int32),
            pltpu.VMEM((CHUNK, hidden), jnp.float32),
            pltpu.VMEM((CHUNK, hidden), jnp.float32),
            pltpu.VMEM((CHUNK, hidden), jnp.float32),
            pltpu.VMEM((max_pos, hidden), jnp.float32),
            pltpu.VMEM((hidden,), jnp.float32),
            pltpu.VMEM((hidden,), jnp.float32),
            pltpu.SemaphoreType.DMA,
            pltpu.SemaphoreType.DMA,
            pltpu.SemaphoreType.DMA,
            pltpu.SemaphoreType.DMA,
            pltpu.SemaphoreType.DMA,
            pltpu.SemaphoreType.DMA,
        ],
    )
    def sc_kernel(ids_hbm, table_hbm, pos_hbm, gam_hbm, bet_hbm, out_hbm,
                  idx_v0, idx_v1, idx_v2, rows_v0, rows_v1, rows_v2,
                  pos_v, gam_v, bet_v,
                  sem0, sem1, sem2, osem0, osem1, osem2):
        wid = lax.axis_index("s") * 2 + lax.axis_index("c")
        base = wid * rpw
        pltpu.sync_copy(pos_hbm, pos_v)
        pltpu.sync_copy(gam_hbm, gam_v)
        pltpu.sync_copy(bet_hbm, bet_v)
        idx_bufs = (idx_v0, idx_v1, idx_v2)
        row_bufs = (rows_v0, rows_v1, rows_v2)
        sems = (sem0, sem1, sem2)
        osems = (osem0, osem1, osem2)
        gs = [gam_v[pl.ds(k * LANES, LANES)] for k in range(nk)]
        bs = [bet_v[pl.ds(k * LANES, LANES)] for k in range(nk)]
        iota16 = lax.iota(jnp.int32, LANES)
        perms = [iota16 ^ d for d in (1, 2, 4, 8)]

        def _lane_sum(v):
            for pm in perms:
                v = v + v.at[pm].get(mode="promise_in_bounds")
            return v

        def start_gather(c, slot):
            r0 = base + c * CHUNK
            pltpu.sync_copy(ids_hbm.at[pl.ds(r0, CHUNK)], idx_bufs[slot])
            pltpu.async_copy(table_hbm.at[idx_bufs[slot]], row_bufs[slot],
                             sems[slot])

        def process(c, slot):
            r0 = base + c * CHUNK
            rows_v = row_bufs[slot]
            pltpu.make_async_copy(table_hbm.at[idx_bufs[slot]], rows_v,
                                  sems[slot]).wait()

            @plsc.parallel_loop(0, CHUNK, unroll=UNROLL)
            def row_body(r):
                p = lax.rem(r0 + r, seq)
                xs = []
                for k in range(nk):
                    t = rows_v[r, pl.ds(k * LANES, LANES)]
                    pe = pos_v[p, pl.ds(k * LANES, LANES)]
                    xs.append(t + pe)
                s = (xs[0] + xs[1]) + (xs[2] + xs[3])
                sq = (xs[0] * xs[0] + xs[1] * xs[1]) + \
                     (xs[2] * xs[2] + xs[3] * xs[3])
                mean = _lane_sum(s) * inv_h
                var = _lane_sum(sq) * inv_h - mean * mean
                rstd = _rsqrt_vec2(var + EPS)
                nm = mean * rstd
                for k in range(nk):
                    y = (xs[k] * rstd - nm) * gs[k] + bs[k]
                    rows_v[r, pl.ds(k * LANES, LANES)] = y

            pltpu.async_copy(rows_v, out_hbm.at[pl.ds(r0, CHUNK)], osems[slot])

        def wait_out(c, slot):
            r0 = base + c * CHUNK
            pltpu.make_async_copy(row_bufs[slot],
                                  out_hbm.at[pl.ds(r0, CHUNK)],
                                  osems[slot]).wait()

        start_gather(jnp.int32(0), 0)
        start_gather(jnp.int32(1), 1)

        def triple_body(i, carry):
            for u in range(3):
                c = 3 * i + u
                nslot = (u + 2) % 3
                process(c, u)

                @pl.when(c >= 1)
                def _():
                    wait_out(c - 1, nslot)

                start_gather(c + 2, nslot)
            return carry

        n_main = (n_chunks - 2) // 3 * 3
        lax.fori_loop(0, n_main // 3, triple_body, 0)
        for c in range(n_main, n_chunks):
            process(c, c % 3)
            wait_out(c - 1, (c - 1) % 3)
        wait_out(n_chunks - 1, (n_chunks - 1) % 3)

    return sc_kernel


def kernel(input_ids, token_table, pos_table, ln_gamma, ln_beta):
    b, s = input_ids.shape
    vocab, hidden = token_table.shape
    max_pos = pos_table.shape[0]
    rows = b * s
    ids = input_ids.reshape(rows).astype(jnp.int32)
    sc = _make_sc_kernel(rows, hidden, max_pos, s)
    out = sc(ids, token_table, pos_table, ln_gamma, ln_beta)
    return out.reshape(b, s, hidden)


# speedup vs baseline: 1.1725x; 1.0099x over previous
import functools

import jax
import jax.numpy as jnp
from jax import lax
from jax.experimental import pallas as pl
from jax.experimental.pallas import tpu as pltpu
from jax.experimental.pallas import tpu_sc as plsc

LANES = 16
CHUNK = 128
UNROLL = 4
EPS = 1e-5


def _rsqrt_vec(x):
    i = plsc.bitcast(x, jnp.int32)
    i = jnp.int32(0x5F3759DF) - lax.shift_right_logical(i, 1)
    y = plsc.bitcast(i, jnp.float32)
    hx = 0.5 * x
    for _ in range(2):
        y = y * (1.5 - hx * y * y)
    return y


def _make_sc_kernel(rows, hidden, max_pos, seq):
    n_workers = 32
    rpw = rows // n_workers
    n_chunks = rpw // CHUNK
    nk = hidden // LANES
    inv_h = 1.0 / hidden

    mesh = plsc.VectorSubcoreMesh(core_axis_name="c", subcore_axis_name="s")

    @functools.partial(
        pl.kernel,
        mesh=mesh,
        out_type=jax.ShapeDtypeStruct((rows, hidden), jnp.float32),
        compiler_params=pltpu.CompilerParams(
            needs_layout_passes=False, use_tc_tiling_on_sc=False),
        scratch_types=[
            pltpu.VMEM((CHUNK,), jnp.int32),
            pltpu.VMEM((CHUNK,), jnp.int32),
            pltpu.VMEM((CHUNK,), jnp.int32),
            pltpu.VMEM((CHUNK, hidden), jnp.float32),
            pltpu.VMEM((CHUNK, hidden), jnp.float32),
            pltpu.VMEM((CHUNK, hidden), jnp.float32),
            pltpu.VMEM((max_pos, hidden), jnp.float32),
            pltpu.VMEM((hidden,), jnp.float32),
            pltpu.VMEM((hidden,), jnp.float32),
            pltpu.SemaphoreType.DMA,
            pltpu.SemaphoreType.DMA,
            pltpu.SemaphoreType.DMA,
            pltpu.SemaphoreType.DMA,
            pltpu.SemaphoreType.DMA,
            pltpu.SemaphoreType.DMA,
        ],
    )
    def sc_kernel(ids_hbm, table_hbm, pos_hbm, gam_hbm, bet_hbm, out_hbm,
                  idx_v0, idx_v1, idx_v2, rows_v0, rows_v1, rows_v2,
                  pos_v, gam_v, bet_v,
                  sem0, sem1, sem2, osem0, osem1, osem2):
        wid = lax.axis_index("s") * 2 + lax.axis_index("c")
        base = wid * rpw
        pltpu.sync_copy(pos_hbm, pos_v)
        pltpu.sync_copy(gam_hbm, gam_v)
        pltpu.sync_copy(bet_hbm, bet_v)
        idx_bufs = (idx_v0, idx_v1, idx_v2)
        row_bufs = (rows_v0, rows_v1, rows_v2)
        sems = (sem0, sem1, sem2)
        osems = (osem0, osem1, osem2)
        gs = [gam_v[pl.ds(k * LANES, LANES)] for k in range(nk)]
        bs = [bet_v[pl.ds(k * LANES, LANES)] for k in range(nk)]
        iota16 = lax.iota(jnp.int32, LANES)
        perms = [iota16 ^ d for d in (1, 2, 4, 8)]

        def _lane_sum(v):
            for pm in perms:
                v = v + v.at[pm].get(mode="promise_in_bounds")
            return v

        def start_gather(c, slot):
            r0 = base + c * CHUNK
            pltpu.sync_copy(ids_hbm.at[pl.ds(r0, CHUNK)], idx_bufs[slot])
            pltpu.async_copy(table_hbm.at[idx_bufs[slot]], row_bufs[slot],
                             sems[slot])

        def process(c, slot):
            r0 = base + c * CHUNK
            rows_v = row_bufs[slot]
            pltpu.make_async_copy(table_hbm.at[idx_bufs[slot]], rows_v,
                                  sems[slot]).wait()

            @plsc.parallel_loop(0, CHUNK, unroll=UNROLL)
            def row_body(r):
                p = lax.rem(r0 + r, seq)
                xs = []
                for k in range(nk):
                    t = rows_v[r, pl.ds(k * LANES, LANES)]
                    pe = pos_v[p, pl.ds(k * LANES, LANES)]
                    xs.append(t + pe)
                s = (xs[0] + xs[1]) + (xs[2] + xs[3])
                sq = (xs[0] * xs[0] + xs[1] * xs[1]) + \
                     (xs[2] * xs[2] + xs[3] * xs[3])
                mean = _lane_sum(s) * inv_h
                var = _lane_sum(sq) * inv_h - mean * mean
                rstd = _rsqrt_vec(var + EPS)
                nm = mean * rstd
                for k in range(nk):
                    y = (xs[k] * rstd - nm) * gs[k] + bs[k]
                    rows_v[r, pl.ds(k * LANES, LANES)] = y

            pltpu.async_copy(rows_v, out_hbm.at[pl.ds(r0, CHUNK)], osems[slot])

        def wait_out(c, slot):
            r0 = base + c * CHUNK
            pltpu.make_async_copy(row_bufs[slot],
                                  out_hbm.at[pl.ds(r0, CHUNK)],
                                  osems[slot]).wait()

        start_gather(jnp.int32(0), 0)
        start_gather(jnp.int32(1), 1)

        def triple_body(i, carry):
            for u in range(3):
                c = 3 * i + u
                nslot = (u + 2) % 3
                process(c, u)

                @pl.when(c >= 1)
                def _():
                    wait_out(c - 1, nslot)

                start_gather(c + 2, nslot)
            return carry

        n_main = (n_chunks - 2) // 3 * 3
        lax.fori_loop(0, n_main // 3, triple_body, 0)
        for c in range(n_main, n_chunks):
            process(c, c % 3)
            wait_out(c - 1, (c - 1) % 3)
        wait_out(n_chunks - 1, (n_chunks - 1) % 3)

    return sc_kernel


def kernel(input_ids, token_table, pos_table, ln_gamma, ln_beta):
    b, s = input_ids.shape
    vocab, hidden = token_table.shape
    max_pos = pos_table.shape[0]
    rows = b * s
    ids = input_ids.reshape(rows).astype(jnp.int32)
    sc = _make_sc_kernel(rows, hidden, max_pos, s)
    out = sc(ids, token_table, pos_table, ln_gamma, ln_beta)
    return out.reshape(b, s, hidden)
